# baseline (speedup 1.0000x reference)
import jax
import jax.numpy as jnp
from jax.experimental import pallas as pl
from jax.experimental.pallas import tpu as pltpu

_B, _N, _C = 4, 4096, 64
_M, _K = 1024, 32
_HI = jax.lax.Precision.HIGHEST
_BF = jnp.bfloat16
_F32 = jnp.float32


def _r(x):
    return x.astype(_BF).astype(_F32)


def _fps_body(x_ref, xt_ref, idx_ref, cx_ref):
    iota_n = jax.lax.broadcasted_iota(jnp.int32, (1, _N), 1)
    iota_m = jax.lax.broadcasted_iota(jnp.int32, (1, _M), 1)

    def step(i, carry):
        fars, dists, accs = carry
        new_fars, new_dists, new_accs = [], [], []
        for b in range(_B):
            far = fars[b]
            c = x_ref[b, pl.ds(far, 1), :]
            cx_ref[b, pl.ds(i, 1), :] = c
            d = jnp.zeros((1, _N), _F32)
            for cc in range(_C):
                t = xt_ref[b, pl.ds(cc, 1), :] - c[0:1, cc:cc + 1]
                d = d + t * t
            dist = jnp.minimum(dists[b], d)
            m = jnp.max(dist)
            nf = jnp.min(jnp.where(dist == m, iota_n, _N)).astype(jnp.int32)
            acc = jnp.where(iota_m == i, far, accs[b])
            new_fars.append(nf)
            new_dists.append(dist)
            new_accs.append(acc)
        return (tuple(new_fars), tuple(new_dists), tuple(new_accs))

    fars0 = tuple(jnp.int32(0) for _ in range(_B))
    dists0 = tuple(jnp.full((1, _N), 1e10, _F32) for _ in range(_B))
    accs0 = tuple(jnp.zeros((1, _M), jnp.int32) for _ in range(_B))
    fars, dists, accs = jax.lax.fori_loop(0, _M, step, (fars0, dists0, accs0))
    for b in range(_B):
        idx_ref[pl.ds(b, 1), :] = accs[b]


def _run_fps(xyz):
    xt = jnp.swapaxes(xyz, 1, 2)
    return pl.pallas_call(
        _fps_body,
        out_shape=[
            jax.ShapeDtypeStruct((_B, _M), jnp.int32),
            jax.ShapeDtypeStruct((_B, _M, _C), _F32),
        ],
        in_specs=[pl.BlockSpec(memory_space=pltpu.VMEM),
                  pl.BlockSpec(memory_space=pltpu.VMEM)],
        out_specs=[
            pl.BlockSpec(memory_space=pltpu.VMEM),
            pl.BlockSpec(memory_space=pltpu.VMEM),
        ],
    )(xyz, xt)


_CB = 256


def _knn_body(x_ref, cx_ref, xt_ref, mask_ref):
    x = x_ref[0]
    xt = xt_ref[0]
    cx = cx_ref[0]
    sq = jnp.sum(x * x, axis=-1).reshape(1, _N)
    csq = jnp.sum(cx * cx, axis=-1).reshape(_CB, 1)
    cross = jax.lax.dot_general(
        _r(cx), _r(xt), (((1,), (0,)), ((), ())),
        precision=_HI, preferred_element_type=_F32)
    d = csq - 2.0 * cross + sq
    iota = jax.lax.broadcasted_iota(jnp.int32, (_CB, _N), 1)
    macc = jnp.zeros((_CB, _N), jnp.int32)
    for _ in range(_K):
        m = jnp.min(d, axis=1, keepdims=True)
        amin = jnp.min(jnp.where(d <= m, iota, _N), axis=1, keepdims=True)
        hit = iota == amin
        macc = jnp.where(hit, 1, macc)
        d = jnp.where(hit, jnp.float32(jnp.inf), d)
    mask_ref[0] = macc.astype(jnp.int8)


def _run_knn(cx, xyz):
    xt = jnp.swapaxes(xyz, 1, 2)
    return pl.pallas_call(
        _knn_body,
        grid=(_B, _M // _CB),
        out_shape=jax.ShapeDtypeStruct((_B, _M, _N), jnp.int8),
        in_specs=[
            pl.BlockSpec((1, _N, _C), lambda b, cb: (b, 0, 0)),
            pl.BlockSpec((1, _CB, _C), lambda b, cb: (b, cb, 0)),
            pl.BlockSpec((1, _C, _N), lambda b, cb: (b, 0, 0)),
        ],
        out_specs=pl.BlockSpec((1, _CB, _N), lambda b, cb: (b, cb, 0)),
    )(xyz, cx, xt)


def _tables_body(f_ref, wq_ref, wk_ref, wv_ref, q_ref, k_ref, v_ref):
    f = _r(f_ref[0, 0])
    for w_ref, o_ref in ((wq_ref, q_ref), (wk_ref, k_ref), (wv_ref, v_ref)):
        w = _r(w_ref[0])
        o = jax.lax.dot_general(f, w, (((1,), (0,)), ((), ())),
                                precision=_HI, preferred_element_type=_F32)
        o_ref[0, 0] = o.astype(_BF)


def _run_tables(F, WQ, WK, WV):
    ns = F.shape[0]
    tbl = jax.ShapeDtypeStruct((ns, _B, _N, _C), _BF)
    return pl.pallas_call(
        _tables_body,
        grid=(ns, _B),
        out_shape=[tbl, tbl, tbl],
        in_specs=[
            pl.BlockSpec((1, 1, _N, _C), lambda s, b: (s, b, 0, 0)),
            pl.BlockSpec((1, _C, _C), lambda s, b: (s, 0, 0)),
            pl.BlockSpec((1, _C, _C), lambda s, b: (s, 0, 0)),
            pl.BlockSpec((1, _C, _C), lambda s, b: (s, 0, 0)),
        ],
        out_specs=[
            pl.BlockSpec((1, 1, _N, _C), lambda s, b: (s, b, 0, 0)),
            pl.BlockSpec((1, 1, _N, _C), lambda s, b: (s, b, 0, 0)),
            pl.BlockSpec((1, 1, _N, _C), lambda s, b: (s, b, 0, 0)),
        ],
    )(F, WQ, WK, WV)


_AB = 256


def _attn_body(q_ref, k_ref, v_ref, mask_ref, fps_ref,
               w1_ref, b1_ref, w2_ref, b2_ref, out_ref):
    fi = fps_ref[0]
    iota_n = jax.lax.broadcasted_iota(jnp.int32, (_AB, _N), 1)
    onehot = (iota_n == fi).astype(_F32)
    qt = q_ref[0, 0].astype(_F32)
    q = jax.lax.dot_general(onehot, qt, (((1,), (0,)), ((), ())),
                            precision=_HI, preferred_element_type=_F32)
    kt = k_ref[0, 0].astype(_F32)
    s = jax.lax.dot_general(q, kt, (((1,), (1,)), ((), ())),
                            precision=_HI, preferred_element_type=_F32)
    lm = s * 0.125
    msk = mask_ref[0] != 0
    mmax = jnp.max(jnp.where(msk, lm, -1e30), axis=1, keepdims=True)
    p = jnp.where(msk, jnp.exp(lm - mmax), 0.0)
    den = jnp.sum(p, axis=1, keepdims=True)
    attn = _r(p / den)
    vt = v_ref[0, 0].astype(_F32)
    a = jax.lax.dot_general(attn, vt, (((1,), (0,)), ((), ())),
                            precision=_HI, preferred_element_type=_F32)
    h = jax.lax.dot_general(_r(a), _r(w1_ref[0]), (((1,), (0,)), ((), ())),
                            precision=_HI, preferred_element_type=_F32)
    h = jnp.maximum(h + b1_ref[0], 0.0)
    o = jax.lax.dot_general(_r(h), _r(w2_ref[0]), (((1,), (0,)), ((), ())),
                            precision=_HI, preferred_element_type=_F32)
    out_ref[0, 0] = jnp.maximum(o + b2_ref[0], 0.0)


def _run_attn(Q, K, V, mask, fps_col, W1, B1, W2, B2):
    ns = Q.shape[0]
    co = W2.shape[-1]
    return pl.pallas_call(
        _attn_body,
        grid=(_B, ns, _M // _AB),
        out_shape=jax.ShapeDtypeStruct((ns, _B, _M, co), _F32),
        in_specs=[
            pl.BlockSpec((1, 1, _N, _C), lambda b, s, mb: (s, b, 0, 0)),
            pl.BlockSpec((1, 1, _N, _C), lambda b, s, mb: (s, b, 0, 0)),
            pl.BlockSpec((1, 1, _N, _C), lambda b, s, mb: (s, b, 0, 0)),
            pl.BlockSpec((1, _AB, _N), lambda b, s, mb: (b, mb, 0)),
            pl.BlockSpec((1, _AB, 1), lambda b, s, mb: (b, mb, 0)),
            pl.BlockSpec((1, _C, co), lambda b, s, mb: (s, 0, 0)),
            pl.BlockSpec((1, 1, co), lambda b, s, mb: (s, 0, 0)),
            pl.BlockSpec((1, co, co), lambda b, s, mb: (s, 0, 0)),
            pl.BlockSpec((1, 1, co), lambda b, s, mb: (s, 0, 0)),
        ],
        out_specs=pl.BlockSpec((1, 1, _AB, co), lambda b, s, mb: (s, b, mb, 0)),
    )(Q, K, V, mask, fps_col, W1, B1, W2, B2)


def _fps_jnp(xyz, m):
    b, n, c = xyz.shape
    centroids = jnp.zeros((b, m), dtype=jnp.int32)
    distance = jnp.full((b, n), 1e10, dtype=xyz.dtype)
    farthest = jnp.zeros((b,), dtype=jnp.int32)

    def body(i, carry):
        centroids, distance, farthest = carry
        centroids = centroids.at[:, i].set(farthest)
        centroid = xyz[jnp.arange(b), farthest]
        d = jnp.sum((xyz - centroid[:, None, :]) ** 2, axis=-1)
        distance = jnp.minimum(distance, d)
        farthest = jnp.argmax(distance, axis=-1).astype(jnp.int32)
        return (centroids, distance, farthest)

    centroids, _, _ = jax.lax.fori_loop(0, m, body,
                                        (centroids, distance, farthest))
    return centroids


def kernel(xyz_fea, pmt_fea, mad_fea, dim_fea, nor_fea, loc_fea, fea, params):
    xyz = xyz_fea
    fps_idx = _fps_jnp(xyz, _M)

    def _ip(points, idx):
        nb = points.shape[0]
        bi = jnp.arange(nb).reshape((nb,) + (1,) * (idx.ndim - 1))
        return points[bi, idx]

    cx = _ip(xyz, fps_idx)
    mask = _run_knn(cx, xyz)

    feats = (xyz_fea, pmt_fea, mad_fea, dim_fea, nor_fea, loc_fea, fea)
    streams = ('xyz', 'pmt', 'mad', 'dim', 'nor', 'loc', 'fea')
    F = jnp.stack(feats)
    WQ = jnp.stack([params[s]['Wq'] for s in streams])
    WK = jnp.stack([params[s]['Wk'] for s in streams])
    WV = jnp.stack([params[s]['Wv'] for s in streams])
    W1 = jnp.stack([params[s]['W1'] for s in streams])
    B1 = jnp.stack([params[s]['b1'] for s in streams])[:, None, :]
    W2 = jnp.stack([params[s]['W2'] for s in streams])
    B2 = jnp.stack([params[s]['b2'] for s in streams])[:, None, :]

    Q, K, V = _run_tables(F, WQ, WK, WV)
    O = _run_attn(Q, K, V, mask, fps_idx[:, :, None], W1, B1, W2, B2)
    return tuple(O[i] for i in range(7))


# speedup vs baseline: 1.0011x; 1.0011x over previous
import jax
import jax.numpy as jnp
from jax.experimental import pallas as pl
from jax.experimental.pallas import tpu as pltpu

_B, _N, _C = 4, 4096, 64
_M, _K = 1024, 32
_HI = jax.lax.Precision.HIGHEST
_BF = jnp.bfloat16
_F32 = jnp.float32


def _r(x):
    return x.astype(_BF).astype(_F32)


def _fps_body(x_ref, xt_ref, idx_ref, cx_ref):
    iota_n = jax.lax.broadcasted_iota(jnp.int32, (1, _N), 1)
    iota_m = jax.lax.broadcasted_iota(jnp.int32, (1, _M), 1)

    x3 = x_ref[...]

    def step(i, carry):
        fars, dists, accs = carry
        cs = []
        for b in range(_B):
            c = x_ref[b, pl.ds(fars[b], 1), :]
            cx_ref[b, pl.ds(i, 1), :] = c
            cs.append(c)
        c3 = jnp.concatenate(cs, axis=0)[:, None, :]
        d3 = jnp.sum((x3 - c3) ** 2, axis=-1)
        new_fars, new_dists, new_accs = [], [], []
        for b in range(_B):
            dist = jnp.minimum(dists[b], d3[b:b + 1, :])
            m = jnp.max(dist)
            nf = jnp.min(jnp.where(dist == m, iota_n, _N)).astype(jnp.int32)
            acc = jnp.where(iota_m == i, fars[b], accs[b])
            new_fars.append(nf)
            new_dists.append(dist)
            new_accs.append(acc)
        return (tuple(new_fars), tuple(new_dists), tuple(new_accs))

    fars0 = tuple(jnp.int32(0) for _ in range(_B))
    dists0 = tuple(jnp.full((1, _N), 1e10, _F32) for _ in range(_B))
    accs0 = tuple(jnp.zeros((1, _M), jnp.int32) for _ in range(_B))
    fars, dists, accs = jax.lax.fori_loop(0, _M, step, (fars0, dists0, accs0))
    for b in range(_B):
        idx_ref[pl.ds(b, 1), :] = accs[b]


def _run_fps(xyz):
    xt = jnp.swapaxes(xyz, 1, 2)
    return pl.pallas_call(
        _fps_body,
        out_shape=[
            jax.ShapeDtypeStruct((_B, _M), jnp.int32),
            jax.ShapeDtypeStruct((_B, _M, _C), _F32),
        ],
        in_specs=[pl.BlockSpec(memory_space=pltpu.VMEM),
                  pl.BlockSpec(memory_space=pltpu.VMEM)],
        out_specs=[
            pl.BlockSpec(memory_space=pltpu.VMEM),
            pl.BlockSpec(memory_space=pltpu.VMEM),
        ],
    )(xyz, xt)


_CB = 256


def _knn_body(x_ref, cx_ref, xt_ref, mask_ref):
    x = x_ref[0]
    xt = xt_ref[0]
    cx = cx_ref[0]
    sq = jnp.sum(x * x, axis=-1).reshape(1, _N)
    csq = jnp.sum(cx * cx, axis=-1).reshape(_CB, 1)
    cross = jax.lax.dot_general(
        _r(cx), _r(xt), (((1,), (0,)), ((), ())),
        precision=_HI, preferred_element_type=_F32)
    d = csq - 2.0 * cross + sq
    iota = jax.lax.broadcasted_iota(jnp.int32, (_CB, _N), 1)
    macc = jnp.zeros((_CB, _N), jnp.int32)
    for _ in range(_K):
        m = jnp.min(d, axis=1, keepdims=True)
        amin = jnp.min(jnp.where(d <= m, iota, _N), axis=1, keepdims=True)
        hit = iota == amin
        macc = jnp.where(hit, 1, macc)
        d = jnp.where(hit, jnp.float32(jnp.inf), d)
    mask_ref[0] = macc.astype(jnp.int8)


def _run_knn(cx, xyz):
    xt = jnp.swapaxes(xyz, 1, 2)
    return pl.pallas_call(
        _knn_body,
        grid=(_B, _M // _CB),
        out_shape=jax.ShapeDtypeStruct((_B, _M, _N), jnp.int8),
        in_specs=[
            pl.BlockSpec((1, _N, _C), lambda b, cb: (b, 0, 0)),
            pl.BlockSpec((1, _CB, _C), lambda b, cb: (b, cb, 0)),
            pl.BlockSpec((1, _C, _N), lambda b, cb: (b, 0, 0)),
        ],
        out_specs=pl.BlockSpec((1, _CB, _N), lambda b, cb: (b, cb, 0)),
    )(xyz, cx, xt)


def _tables_body(f_ref, wq_ref, wk_ref, wv_ref, q_ref, k_ref, v_ref):
    f = _r(f_ref[0, 0])
    for w_ref, o_ref in ((wq_ref, q_ref), (wk_ref, k_ref), (wv_ref, v_ref)):
        w = _r(w_ref[0])
        o = jax.lax.dot_general(f, w, (((1,), (0,)), ((), ())),
                                precision=_HI, preferred_element_type=_F32)
        o_ref[0, 0] = o.astype(_BF)


def _run_tables(F, WQ, WK, WV):
    ns = F.shape[0]
    tbl = jax.ShapeDtypeStruct((ns, _B, _N, _C), _BF)
    return pl.pallas_call(
        _tables_body,
        grid=(ns, _B),
        out_shape=[tbl, tbl, tbl],
        in_specs=[
            pl.BlockSpec((1, 1, _N, _C), lambda s, b: (s, b, 0, 0)),
            pl.BlockSpec((1, _C, _C), lambda s, b: (s, 0, 0)),
            pl.BlockSpec((1, _C, _C), lambda s, b: (s, 0, 0)),
            pl.BlockSpec((1, _C, _C), lambda s, b: (s, 0, 0)),
        ],
        out_specs=[
            pl.BlockSpec((1, 1, _N, _C), lambda s, b: (s, b, 0, 0)),
            pl.BlockSpec((1, 1, _N, _C), lambda s, b: (s, b, 0, 0)),
            pl.BlockSpec((1, 1, _N, _C), lambda s, b: (s, b, 0, 0)),
        ],
    )(F, WQ, WK, WV)


_AB = 256


def _attn_body(q_ref, k_ref, v_ref, mask_ref, fps_ref,
               w1_ref, b1_ref, w2_ref, b2_ref, out_ref):
    fi = fps_ref[0]
    iota_n = jax.lax.broadcasted_iota(jnp.int32, (_AB, _N), 1)
    onehot = (iota_n == fi).astype(_F32)
    qt = q_ref[0, 0].astype(_F32)
    q = jax.lax.dot_general(onehot, qt, (((1,), (0,)), ((), ())),
                            precision=_HI, preferred_element_type=_F32)
    kt = k_ref[0, 0].astype(_F32)
    s = jax.lax.dot_general(q, kt, (((1,), (1,)), ((), ())),
                            precision=_HI, preferred_element_type=_F32)
    lm = s * 0.125
    msk = mask_ref[0] != 0
    mmax = jnp.max(jnp.where(msk, lm, -1e30), axis=1, keepdims=True)
    p = jnp.where(msk, jnp.exp(lm - mmax), 0.0)
    den = jnp.sum(p, axis=1, keepdims=True)
    attn = _r(p / den)
    vt = v_ref[0, 0].astype(_F32)
    a = jax.lax.dot_general(attn, vt, (((1,), (0,)), ((), ())),
                            precision=_HI, preferred_element_type=_F32)
    h = jax.lax.dot_general(_r(a), _r(w1_ref[0]), (((1,), (0,)), ((), ())),
                            precision=_HI, preferred_element_type=_F32)
    h = jnp.maximum(h + b1_ref[0], 0.0)
    o = jax.lax.dot_general(_r(h), _r(w2_ref[0]), (((1,), (0,)), ((), ())),
                            precision=_HI, preferred_element_type=_F32)
    out_ref[0, 0] = jnp.maximum(o + b2_ref[0], 0.0)


def _run_attn(Q, K, V, mask, fps_col, W1, B1, W2, B2):
    ns = Q.shape[0]
    co = W2.shape[-1]
    return pl.pallas_call(
        _attn_body,
        grid=(_B, ns, _M // _AB),
        out_shape=jax.ShapeDtypeStruct((ns, _B, _M, co), _F32),
        in_specs=[
            pl.BlockSpec((1, 1, _N, _C), lambda b, s, mb: (s, b, 0, 0)),
            pl.BlockSpec((1, 1, _N, _C), lambda b, s, mb: (s, b, 0, 0)),
            pl.BlockSpec((1, 1, _N, _C), lambda b, s, mb: (s, b, 0, 0)),
            pl.BlockSpec((1, _AB, _N), lambda b, s, mb: (b, mb, 0)),
            pl.BlockSpec((1, _AB, 1), lambda b, s, mb: (b, mb, 0)),
            pl.BlockSpec((1, _C, co), lambda b, s, mb: (s, 0, 0)),
            pl.BlockSpec((1, 1, co), lambda b, s, mb: (s, 0, 0)),
            pl.BlockSpec((1, co, co), lambda b, s, mb: (s, 0, 0)),
            pl.BlockSpec((1, 1, co), lambda b, s, mb: (s, 0, 0)),
        ],
        out_specs=pl.BlockSpec((1, 1, _AB, co), lambda b, s, mb: (s, b, mb, 0)),
    )(Q, K, V, mask, fps_col, W1, B1, W2, B2)


def _fps_jnp(xyz, m):
    b, n, c = xyz.shape
    centroids = jnp.zeros((b, m), dtype=jnp.int32)
    distance = jnp.full((b, n), 1e10, dtype=xyz.dtype)
    farthest = jnp.zeros((b,), dtype=jnp.int32)

    def body(i, carry):
        centroids, distance, farthest = carry
        centroids = centroids.at[:, i].set(farthest)
        centroid = xyz[jnp.arange(b), farthest]
        d = jnp.sum((xyz - centroid[:, None, :]) ** 2, axis=-1)
        distance = jnp.minimum(distance, d)
        farthest = jnp.argmax(distance, axis=-1).astype(jnp.int32)
        return (centroids, distance, farthest)

    centroids, _, _ = jax.lax.fori_loop(0, m, body,
                                        (centroids, distance, farthest))
    return centroids


def kernel(xyz_fea, pmt_fea, mad_fea, dim_fea, nor_fea, loc_fea, fea, params):
    xyz = xyz_fea
    fps_idx = _fps_jnp(xyz, _M)

    def _ip(points, idx):
        nb = points.shape[0]
        bi = jnp.arange(nb).reshape((nb,) + (1,) * (idx.ndim - 1))
        return points[bi, idx]

    cx = _ip(xyz, fps_idx)
    mask = _run_knn(cx, xyz)

    feats = (xyz_fea, pmt_fea, mad_fea, dim_fea, nor_fea, loc_fea, fea)
    streams = ('xyz', 'pmt', 'mad', 'dim', 'nor', 'loc', 'fea')
    F = jnp.stack(feats)
    WQ = jnp.stack([params[s]['Wq'] for s in streams])
    WK = jnp.stack([params[s]['Wk'] for s in streams])
    WV = jnp.stack([params[s]['Wv'] for s in streams])
    W1 = jnp.stack([params[s]['W1'] for s in streams])
    B1 = jnp.stack([params[s]['b1'] for s in streams])[:, None, :]
    W2 = jnp.stack([params[s]['W2'] for s in streams])
    B2 = jnp.stack([params[s]['b2'] for s in streams])[:, None, :]

    Q, K, V = _run_tables(F, WQ, WK, WV)
    O = _run_attn(Q, K, V, mask, fps_idx[:, :, None], W1, B1, W2, B2)
    return tuple(O[i] for i in range(7))
